# baseline (speedup 1.0000x reference)
import functools

import jax
import jax.numpy as jnp
from jax import lax
from jax.experimental import pallas as pl
from jax.experimental.pallas import tpu as pltpu
from jax.experimental.pallas import tpu_sc as plsc

B, L, D = 2, 2048, 64
G = 32
NH = 4
BW = 4.0
NB = 64
K = 64
BQ = 256
NEG = float(jnp.finfo(jnp.float32).min)

NCORE, NSUB = 2, 16
TPB = NSUB
KPT = L // TPB
QPT = L // TPB
MAPB = 14
MAPW = 1 << MAPB
MMASK = MAPW - 1


def _sc_screen_body(qf_hbm, kf_hbm, out_hbm,
                    feat_v, cb0_v, cb1_v, kc0_v, kc1_v,
                    cm0, cm1, ct0, ct1, flags_v, shared, sem, sem2):
    b = lax.axis_index("c")
    s = lax.axis_index("s")
    base = pl.multiple_of(b * L + s * KPT, KPT)

    dma_k = pltpu.make_async_copy(kf_hbm.at[pl.ds(base, KPT)],
                                  feat_v.at[pl.ds(0, KPT)], sem)
    dma_q = pltpu.make_async_copy(qf_hbm.at[pl.ds(base, QPT)],
                                  feat_v.at[pl.ds(KPT, QPT)], sem2)
    dma_k.start()
    dma_q.start()

    lanes = lax.iota(jnp.int32, 16)
    dma_k.wait()
    dma_q.wait()

    def codes(i, _):
        rows = lanes + i * 16
        acc0 = jnp.zeros((16,), jnp.int32)
        acc1 = jnp.zeros((16,), jnp.int32)
        for d in range(G):
            cols = jnp.full((16,), d, jnp.int32)
            v0 = plsc.load_gather(feat_v, [rows, cols])
            v1 = plsc.load_gather(feat_v, [rows, cols + G])
            acc0 = acc0 | ((v0 > 0).astype(jnp.int32) << d)
            acc1 = acc1 | ((v1 > 0).astype(jnp.int32) << d)
        off = pl.multiple_of(i * 16, 16)
        cb0_v[pl.ds(off, 16)] = acc0
        cb1_v[pl.ds(off, 16)] = acc1
        return 0

    lax.fori_loop(0, (KPT + QPT) // 16, codes, 0)

    soff = pl.multiple_of(s * KPT, KPT)
    pltpu.sync_copy(cb0_v.at[pl.ds(0, KPT)], shared.at[0, pl.ds(soff, KPT)])
    pltpu.sync_copy(cb1_v.at[pl.ds(0, KPT)], shared.at[1, pl.ds(soff, KPT)])
    plsc.subcore_barrier()
    pltpu.sync_copy(shared.at[0], kc0_v)
    pltpu.sync_copy(shared.at[1], kc1_v)

    UN = 4

    def build(c, _):
        for u in range(UN):
            off = pl.multiple_of(c * 16 * UN + u * 16, 16)
            k0 = kc0_v[pl.ds(off, 16)]
            k1 = kc1_v[pl.ds(off, 16)]
            plsc.store_scatter(cm0, [k0 & MMASK], k0)
            plsc.store_scatter(cm1, [k1 & MMASK], k1)
        return 0

    lax.fori_loop(0, L // (16 * UN), build, 0)

    ones = jnp.full((16,), 1, jnp.int32)

    def contest(c, _):
        for u in range(UN):
            off = pl.multiple_of(c * 16 * UN + u * 16, 16)
            k0 = kc0_v[pl.ds(off, 16)]
            k1 = kc1_v[pl.ds(off, 16)]
            s0 = k0 & MMASK
            s1 = k1 & MMASK
            lost0 = plsc.load_gather(cm0, [s0]) != k0
            lost1 = plsc.load_gather(cm1, [s1]) != k1
            plsc.store_scatter(ct0, [s0], ones, mask=lost0)
            plsc.store_scatter(ct1, [s1], ones, mask=lost1)
        return 0

    lax.fori_loop(0, L // (16 * UN), contest, 0)

    for v in range(QPT // 16):
        off = KPT + v * 16
        q0 = cb0_v[pl.ds(off, 16)]
        q1 = cb1_v[pl.ds(off, 16)]
        s0 = q0 & MMASK
        s1 = q1 & MMASK
        hit0 = plsc.load_gather(cm0, [s0]) == q0
        hit1 = plsc.load_gather(cm1, [s1]) == q1
        con0 = plsc.load_gather(ct0, [s0]) > 0
        con1 = plsc.load_gather(ct1, [s1]) > 0
        flag = hit0 | hit1
        amb = (~flag) & (con0 | con1)
        flags_v[pl.ds(v * 16, 16)] = flag.astype(jnp.int32)
        n_amb = jnp.sum(amb.astype(jnp.int32))

        @pl.when(n_amb > 0)
        def _verify(v=v, amb=amb):
            def w_cond(carry):
                return jnp.max(carry.astype(jnp.int32)) > 0

            def w_body(carry):
                w = plsc.all_reduce_ffs(carry)
                qloc = jnp.zeros((16,), jnp.int32) + KPT + v * 16 + w
                a0 = plsc.load_gather(cb0_v, [qloc])
                a1 = plsc.load_gather(cb1_v, [qloc])

                def scan(c, acc):
                    for u in range(UN):
                        o = pl.multiple_of(c * 16 * UN + u * 16, 16)
                        acc = (acc | (kc0_v[pl.ds(o, 16)] == a0)
                               | (kc1_v[pl.ds(o, 16)] == a1))
                    return acc

                m = lax.fori_loop(0, L // (16 * UN), scan,
                                  jnp.zeros((16,), jnp.bool_))
                res = (plsc.all_reduce_population_count(m) > 0)
                plsc.store_scatter(flags_v, [qloc - KPT],
                                   res.astype(jnp.int32),
                                   mask=lanes == 0)
                return carry & (lanes != w)

            lax.while_loop(w_cond, w_body, amb)

    pltpu.sync_copy(flags_v, out_hbm.at[pl.ds(base, QPT)])


@functools.lru_cache(maxsize=1)
def _make_sc_screen():
    mesh = plsc.VectorSubcoreMesh(core_axis_name="c", subcore_axis_name="s",
                                  num_cores=NCORE, num_subcores=NSUB)
    return pl.kernel(
        _sc_screen_body,
        out_type=jax.ShapeDtypeStruct((B * L,), jnp.int32),
        mesh=mesh,
        compiler_params=pltpu.CompilerParams(needs_layout_passes=False),
        scratch_types=[
            pltpu.VMEM((KPT + QPT, D), jnp.float32),
            pltpu.VMEM((KPT + QPT,), jnp.int32),
            pltpu.VMEM((KPT + QPT,), jnp.int32),
            pltpu.VMEM((L,), jnp.int32),
            pltpu.VMEM((L,), jnp.int32),
            pltpu.VMEM((MAPW,), jnp.int32),
            pltpu.VMEM((MAPW,), jnp.int32),
            pltpu.VMEM((MAPW,), jnp.int32),
            pltpu.VMEM((MAPW,), jnp.int32),
            pltpu.VMEM((QPT,), jnp.int32),
            pltpu.VMEM_SHARED((2, L), jnp.int32),
            pltpu.SemaphoreType.DMA,
            pltpu.SemaphoreType.DMA,
        ],
    )


def _pack_weights():
    d = lax.broadcasted_iota(jnp.int32, (G, 2), 0)
    c = lax.broadcasted_iota(jnp.int32, (G, 2), 1)
    p = jnp.left_shift(jnp.int32(1), lax.rem(d, 16)).astype(jnp.float32)
    return jnp.where(c == d // 16, p, 0.0)


def _body(qf_ref, kf_ref, p0_ref, p1_ref, cand_ref, vals_ref):
    qf = qf_ref[0]
    kf = kf_ref[0]
    projs = (p0_ref[...], p1_ref[...])

    W = _pack_weights()

    qcodes, kcodes, qhash, khash = [], [], [], []
    for g in range(2):
        qg = qf[:, g * G:(g + 1) * G]
        kg = kf[:, g * G:(g + 1) * G]
        qb = (qg > 0).astype(jnp.float32)
        kb = (kg > 0).astype(jnp.float32)
        qcodes.append(lax.dot_general(qb, W, (((1,), (0,)), ((), ())),
                                      preferred_element_type=jnp.float32))
        kcodes.append(lax.dot_general(W, kb, (((0,), (1,)), ((), ())),
                                      preferred_element_type=jnp.float32))
        qy = lax.dot_general(qg, projs[g], (((1,), (0,)), ((), ())),
                             preferred_element_type=jnp.float32)
        ky = lax.dot_general(projs[g], kg, (((0,), (1,)), ((), ())),
                             preferred_element_type=jnp.float32)
        qh = jnp.floor(qy / BW)
        kh = jnp.floor(ky / BW)
        qhash.append(qh - jnp.floor(qh / NB) * NB)
        khash.append(kh - jnp.floor(kh / NB) * NB)

    code_eq = []
    for g in range(2):
        eq = ((qcodes[g][:, 0:1] == kcodes[g][0:1, :]) &
              (qcodes[g][:, 1:2] == kcodes[g][1:2, :]))
        code_eq.append(eq)
    screen = code_eq[0] | code_eq[1]
    any_match = jnp.sum(screen.astype(jnp.int32)) > 0

    cand_ref[...] = jnp.full((1, BQ, K), -1, dtype=jnp.int32)
    vals_ref[...] = jnp.full((1, BQ, K), NEG, dtype=jnp.float32)

    @pl.when(any_match)
    def _heavy():
        full_mask = jnp.zeros((BQ, L), dtype=jnp.bool_)
        for g in range(2):
            lsh = jnp.zeros((BQ, L), dtype=jnp.bool_)
            for h in range(NH):
                lsh = lsh | (qhash[g][:, h:h + 1] == khash[g][h:h + 1, :])
            full_mask = full_mask | (code_eq[g] & lsh)
        scores = lax.dot_general(qf, kf, (((1,), (1,)), ((), ())),
                                 preferred_element_type=jnp.float32)
        masked = jnp.where(full_mask, scores, NEG)
        iota_k = lax.broadcasted_iota(jnp.int32, (BQ, L), 1)
        iota_c = lax.broadcasted_iota(jnp.int32, (BQ, K), 1)

        def step(j, carry):
            m_vals, out_v, out_i = carry
            mx = jnp.max(m_vals, axis=1, keepdims=True)
            idx = jnp.min(jnp.where(m_vals == mx, iota_k, L),
                          axis=1, keepdims=True)
            col = iota_c == j
            out_v = jnp.where(col, mx, out_v)
            out_i = jnp.where(col, jnp.where(mx > NEG, idx, -1), out_i)
            return jnp.where(iota_k == idx, NEG, m_vals), out_v, out_i

        _, out_v, out_i = lax.fori_loop(
            0, K, step,
            (masked,
             jnp.full((BQ, K), NEG, dtype=jnp.float32),
             jnp.full((BQ, K), -1, dtype=jnp.int32)))
        vals_ref[0] = out_v
        cand_ref[0] = out_i


def _run(qf, kf, p0, p1):
    qt = L // BQ
    grid = (B, qt)
    return pl.pallas_call(
        _body,
        grid=grid,
        in_specs=[
            pl.BlockSpec((1, BQ, D), lambda b, t: (b, t, 0)),
            pl.BlockSpec((1, L, D), lambda b, t: (b, 0, 0)),
            pl.BlockSpec((G, NH), lambda b, t: (0, 0)),
            pl.BlockSpec((G, NH), lambda b, t: (0, 0)),
        ],
        out_specs=[
            pl.BlockSpec((1, BQ, K), lambda b, t: (b, t, 0)),
            pl.BlockSpec((1, BQ, K), lambda b, t: (b, t, 0)),
        ],
        out_shape=[
            jax.ShapeDtypeStruct((B, L, K), jnp.int32),
            jax.ShapeDtypeStruct((B, L, K), jnp.float32),
        ],
    )(qf, kf, p0, p1)


@jax.jit
def _dispatch(qf, kf, p0, p1):
    flags = _make_sc_screen()(qf.reshape(B * L, D), kf.reshape(B * L, D))
    any_match = jnp.sum(flags) > 0
    return lax.cond(
        any_match,
        lambda: _run(qf, kf, p0, p1),
        lambda: (jnp.full((B, L, K), -1, dtype=jnp.int32),
                 jnp.full((B, L, K), NEG, dtype=jnp.float32)))


def kernel(query_features, key_features, head_idx, lsh_proj_g0, lsh_proj_g1):
    cand, vals = _dispatch(query_features, key_features,
                           lsh_proj_g0, lsh_proj_g1)
    return cand, vals


# speedup vs baseline: 1.0108x; 1.0108x over previous
import functools

import jax
import jax.numpy as jnp
from jax import lax
from jax.experimental import pallas as pl
from jax.experimental.pallas import tpu as pltpu
from jax.experimental.pallas import tpu_sc as plsc

B, L, D = 2, 2048, 64
G = 32
NH = 4
BW = 4.0
NB = 64
K = 64
BQ = 256
NEG = float(jnp.finfo(jnp.float32).min)

NCORE, NSUB = 2, 16
TPB = NSUB
KPT = L // TPB
QPT = L // TPB
MAPB = 14
MAPW = 1 << MAPB
MMASK = MAPW - 1


def _sc_screen_body(qf_hbm, kf_hbm, out_hbm,
                    feat_v, cb0_v, cb1_v, kc0_v, kc1_v,
                    cm0, cm1, ct0, ct1, flags_v, shared, sem, sem2):
    b = lax.axis_index("c")
    s = lax.axis_index("s")
    base = pl.multiple_of(b * L + s * KPT, KPT)

    dma_k = pltpu.make_async_copy(kf_hbm.at[pl.ds(base, KPT)],
                                  feat_v.at[pl.ds(0, KPT)], sem)
    dma_q = pltpu.make_async_copy(qf_hbm.at[pl.ds(base, QPT)],
                                  feat_v.at[pl.ds(KPT, QPT)], sem2)
    dma_k.start()
    dma_q.start()

    lanes = lax.iota(jnp.int32, 16)
    dma_k.wait()
    dma_q.wait()

    def codes(i, _):
        rows = lanes + i * 16
        acc0 = jnp.zeros((16,), jnp.int32)
        acc1 = jnp.zeros((16,), jnp.int32)
        for d in range(G):
            sh = (lanes + d) & (G - 1)
            v0 = plsc.load_gather(feat_v, [rows, sh])
            v1 = plsc.load_gather(feat_v, [rows, sh + G])
            acc0 = acc0 | ((v0 > 0).astype(jnp.int32) << sh)
            acc1 = acc1 | ((v1 > 0).astype(jnp.int32) << sh)
        off = pl.multiple_of(i * 16, 16)
        cb0_v[pl.ds(off, 16)] = acc0
        cb1_v[pl.ds(off, 16)] = acc1
        return 0

    with jax.named_scope("sc_codes"):
        lax.fori_loop(0, (KPT + QPT) // 16, codes, 0)

    with jax.named_scope("sc_stage"):
        soff = pl.multiple_of(s * KPT, KPT)
        pltpu.sync_copy(cb0_v.at[pl.ds(0, KPT)], shared.at[0, pl.ds(soff, KPT)])
        pltpu.sync_copy(cb1_v.at[pl.ds(0, KPT)], shared.at[1, pl.ds(soff, KPT)])
        plsc.subcore_barrier()
        pltpu.sync_copy(shared.at[0], kc0_v)
        pltpu.sync_copy(shared.at[1], kc1_v)

    UN = 4

    def build(c, _):
        for u in range(UN):
            off = pl.multiple_of(c * 16 * UN + u * 16, 16)
            k0 = kc0_v[pl.ds(off, 16)]
            k1 = kc1_v[pl.ds(off, 16)]
            plsc.store_scatter(cm0, [k0 & MMASK], k0)
            plsc.store_scatter(cm1, [k1 & MMASK], k1)
        return 0

    with jax.named_scope("sc_build"):
        lax.fori_loop(0, L // (16 * UN), build, 0)

    ones = jnp.full((16,), 1, jnp.int32)

    def contest(c, _):
        for u in range(UN):
            off = pl.multiple_of(c * 16 * UN + u * 16, 16)
            k0 = kc0_v[pl.ds(off, 16)]
            k1 = kc1_v[pl.ds(off, 16)]
            s0 = k0 & MMASK
            s1 = k1 & MMASK
            lost0 = plsc.load_gather(cm0, [s0]) != k0
            lost1 = plsc.load_gather(cm1, [s1]) != k1
            plsc.store_scatter(ct0, [s0], ones, mask=lost0)
            plsc.store_scatter(ct1, [s1], ones, mask=lost1)
        return 0

    with jax.named_scope("sc_contest"):
        lax.fori_loop(0, L // (16 * UN), contest, 0)

    sscope = jax.named_scope("sc_screen")
    sscope.__enter__()
    for v in range(QPT // 16):
        off = KPT + v * 16
        q0 = cb0_v[pl.ds(off, 16)]
        q1 = cb1_v[pl.ds(off, 16)]
        s0 = q0 & MMASK
        s1 = q1 & MMASK
        hit0 = plsc.load_gather(cm0, [s0]) == q0
        hit1 = plsc.load_gather(cm1, [s1]) == q1
        con0 = plsc.load_gather(ct0, [s0]) > 0
        con1 = plsc.load_gather(ct1, [s1]) > 0
        flag = hit0 | hit1
        amb = (~flag) & (con0 | con1)
        flags_v[pl.ds(v * 16, 16)] = flag.astype(jnp.int32)
        n_amb = jnp.sum(amb.astype(jnp.int32))

        @pl.when(n_amb > 0)
        def _verify(v=v, amb=amb):
            def w_cond(carry):
                return jnp.max(carry.astype(jnp.int32)) > 0

            def w_body(carry):
                w = plsc.all_reduce_ffs(carry)
                qloc = jnp.zeros((16,), jnp.int32) + KPT + v * 16 + w
                a0 = plsc.load_gather(cb0_v, [qloc])
                a1 = plsc.load_gather(cb1_v, [qloc])

                def scan(c, acc):
                    for u in range(UN):
                        o = pl.multiple_of(c * 16 * UN + u * 16, 16)
                        acc = (acc | (kc0_v[pl.ds(o, 16)] == a0)
                               | (kc1_v[pl.ds(o, 16)] == a1))
                    return acc

                m = lax.fori_loop(0, L // (16 * UN), scan,
                                  jnp.zeros((16,), jnp.bool_))
                res = (plsc.all_reduce_population_count(m) > 0)
                plsc.store_scatter(flags_v, [qloc - KPT],
                                   res.astype(jnp.int32),
                                   mask=lanes == 0)
                return carry & (lanes != w)

            lax.while_loop(w_cond, w_body, amb)

    sscope.__exit__(None, None, None)
    pltpu.sync_copy(flags_v, out_hbm.at[pl.ds(base, QPT)])


@functools.lru_cache(maxsize=1)
def _make_sc_screen():
    mesh = plsc.VectorSubcoreMesh(core_axis_name="c", subcore_axis_name="s",
                                  num_cores=NCORE, num_subcores=NSUB)
    return pl.kernel(
        _sc_screen_body,
        out_type=jax.ShapeDtypeStruct((B * L,), jnp.int32),
        mesh=mesh,
        compiler_params=pltpu.CompilerParams(needs_layout_passes=False),
        scratch_types=[
            pltpu.VMEM((KPT + QPT, D), jnp.float32),
            pltpu.VMEM((KPT + QPT,), jnp.int32),
            pltpu.VMEM((KPT + QPT,), jnp.int32),
            pltpu.VMEM((L,), jnp.int32),
            pltpu.VMEM((L,), jnp.int32),
            pltpu.VMEM((MAPW,), jnp.int32),
            pltpu.VMEM((MAPW,), jnp.int32),
            pltpu.VMEM((MAPW,), jnp.int32),
            pltpu.VMEM((MAPW,), jnp.int32),
            pltpu.VMEM((QPT,), jnp.int32),
            pltpu.VMEM_SHARED((2, L), jnp.int32),
            pltpu.SemaphoreType.DMA,
            pltpu.SemaphoreType.DMA,
        ],
    )


def _pack_weights():
    d = lax.broadcasted_iota(jnp.int32, (G, 2), 0)
    c = lax.broadcasted_iota(jnp.int32, (G, 2), 1)
    p = jnp.left_shift(jnp.int32(1), lax.rem(d, 16)).astype(jnp.float32)
    return jnp.where(c == d // 16, p, 0.0)


def _body(qf_ref, kf_ref, p0_ref, p1_ref, cand_ref, vals_ref):
    qf = qf_ref[0]
    kf = kf_ref[0]
    projs = (p0_ref[...], p1_ref[...])

    W = _pack_weights()

    qcodes, kcodes, qhash, khash = [], [], [], []
    for g in range(2):
        qg = qf[:, g * G:(g + 1) * G]
        kg = kf[:, g * G:(g + 1) * G]
        qb = (qg > 0).astype(jnp.float32)
        kb = (kg > 0).astype(jnp.float32)
        qcodes.append(lax.dot_general(qb, W, (((1,), (0,)), ((), ())),
                                      preferred_element_type=jnp.float32))
        kcodes.append(lax.dot_general(W, kb, (((0,), (1,)), ((), ())),
                                      preferred_element_type=jnp.float32))
        qy = lax.dot_general(qg, projs[g], (((1,), (0,)), ((), ())),
                             preferred_element_type=jnp.float32)
        ky = lax.dot_general(projs[g], kg, (((0,), (1,)), ((), ())),
                             preferred_element_type=jnp.float32)
        qh = jnp.floor(qy / BW)
        kh = jnp.floor(ky / BW)
        qhash.append(qh - jnp.floor(qh / NB) * NB)
        khash.append(kh - jnp.floor(kh / NB) * NB)

    code_eq = []
    for g in range(2):
        eq = ((qcodes[g][:, 0:1] == kcodes[g][0:1, :]) &
              (qcodes[g][:, 1:2] == kcodes[g][1:2, :]))
        code_eq.append(eq)
    screen = code_eq[0] | code_eq[1]
    any_match = jnp.sum(screen.astype(jnp.int32)) > 0

    cand_ref[...] = jnp.full((1, BQ, K), -1, dtype=jnp.int32)
    vals_ref[...] = jnp.full((1, BQ, K), NEG, dtype=jnp.float32)

    @pl.when(any_match)
    def _heavy():
        full_mask = jnp.zeros((BQ, L), dtype=jnp.bool_)
        for g in range(2):
            lsh = jnp.zeros((BQ, L), dtype=jnp.bool_)
            for h in range(NH):
                lsh = lsh | (qhash[g][:, h:h + 1] == khash[g][h:h + 1, :])
            full_mask = full_mask | (code_eq[g] & lsh)
        scores = lax.dot_general(qf, kf, (((1,), (1,)), ((), ())),
                                 preferred_element_type=jnp.float32)
        masked = jnp.where(full_mask, scores, NEG)
        iota_k = lax.broadcasted_iota(jnp.int32, (BQ, L), 1)
        iota_c = lax.broadcasted_iota(jnp.int32, (BQ, K), 1)

        def step(j, carry):
            m_vals, out_v, out_i = carry
            mx = jnp.max(m_vals, axis=1, keepdims=True)
            idx = jnp.min(jnp.where(m_vals == mx, iota_k, L),
                          axis=1, keepdims=True)
            col = iota_c == j
            out_v = jnp.where(col, mx, out_v)
            out_i = jnp.where(col, jnp.where(mx > NEG, idx, -1), out_i)
            return jnp.where(iota_k == idx, NEG, m_vals), out_v, out_i

        _, out_v, out_i = lax.fori_loop(
            0, K, step,
            (masked,
             jnp.full((BQ, K), NEG, dtype=jnp.float32),
             jnp.full((BQ, K), -1, dtype=jnp.int32)))
        vals_ref[0] = out_v
        cand_ref[0] = out_i


def _run(qf, kf, p0, p1):
    qt = L // BQ
    grid = (B, qt)
    return pl.pallas_call(
        _body,
        grid=grid,
        in_specs=[
            pl.BlockSpec((1, BQ, D), lambda b, t: (b, t, 0)),
            pl.BlockSpec((1, L, D), lambda b, t: (b, 0, 0)),
            pl.BlockSpec((G, NH), lambda b, t: (0, 0)),
            pl.BlockSpec((G, NH), lambda b, t: (0, 0)),
        ],
        out_specs=[
            pl.BlockSpec((1, BQ, K), lambda b, t: (b, t, 0)),
            pl.BlockSpec((1, BQ, K), lambda b, t: (b, t, 0)),
        ],
        out_shape=[
            jax.ShapeDtypeStruct((B, L, K), jnp.int32),
            jax.ShapeDtypeStruct((B, L, K), jnp.float32),
        ],
    )(qf, kf, p0, p1)


@jax.jit
def _dispatch(qf, kf, p0, p1):
    flags = _make_sc_screen()(qf.reshape(B * L, D), kf.reshape(B * L, D))
    any_match = jnp.sum(flags) > 0
    return lax.cond(
        any_match,
        lambda: _run(qf, kf, p0, p1),
        lambda: (jnp.full((B, L, K), -1, dtype=jnp.int32),
                 jnp.full((B, L, K), NEG, dtype=jnp.float32)))


def kernel(query_features, key_features, head_idx, lsh_proj_g0, lsh_proj_g1):
    cand, vals = _dispatch(query_features, key_features,
                           lsh_proj_g0, lsh_proj_g1)
    return cand, vals


# speedup vs baseline: 1.2052x; 1.1923x over previous
import functools

import jax
import jax.numpy as jnp
from jax import lax
from jax.experimental import pallas as pl
from jax.experimental.pallas import tpu as pltpu
from jax.experimental.pallas import tpu_sc as plsc

B, L, D = 2, 2048, 64
G = 32
NH = 4
BW = 4.0
NB = 64
K = 64
BQ = 256
NEG = float(jnp.finfo(jnp.float32).min)

NCORE, NSUB = 2, 16
TPB = NSUB
KPT = L // TPB
QPT = L // TPB
MAPB = 14
MAPW = 1 << MAPB
MMASK = MAPW - 1


def _sc_screen_body(qf_hbm, kf_hbm, zero_hbm, out_hbm,
                    feat_v, cb0_v, cb1_v, kc0_v, kc1_v,
                    cm0, cm1, ct0, ct1, flags_v, shared, sem, sem2, sem3, sem4):
    b = lax.axis_index("c")
    s = lax.axis_index("s")
    base = pl.multiple_of(b * L + s * KPT, KPT)

    dma_k = pltpu.make_async_copy(kf_hbm.at[pl.ds(base, KPT)],
                                  feat_v.at[pl.ds(0, KPT)], sem)
    dma_q = pltpu.make_async_copy(qf_hbm.at[pl.ds(base, QPT)],
                                  feat_v.at[pl.ds(KPT, QPT)], sem2)
    dma_z0 = pltpu.make_async_copy(zero_hbm, ct0, sem3)
    dma_z1 = pltpu.make_async_copy(zero_hbm, ct1, sem4)
    dma_k.start()
    dma_q.start()
    dma_z0.start()
    dma_z1.start()

    lanes = lax.iota(jnp.int32, 16)
    dma_k.wait()
    dma_q.wait()

    def codes(i, _):
        rows = lanes + i * 16
        acc0 = jnp.zeros((16,), jnp.int32)
        acc1 = jnp.zeros((16,), jnp.int32)
        for d in range(G):
            sh = (lanes + d) & (G - 1)
            v0 = plsc.load_gather(feat_v, [rows, sh])
            v1 = plsc.load_gather(feat_v, [rows, sh + G])
            acc0 = acc0 | ((v0 > 0).astype(jnp.int32) << sh)
            acc1 = acc1 | ((v1 > 0).astype(jnp.int32) << sh)
        off = pl.multiple_of(i * 16, 16)
        cb0_v[pl.ds(off, 16)] = acc0
        cb1_v[pl.ds(off, 16)] = acc1
        return 0

    with jax.named_scope("sc_codes"):
        lax.fori_loop(0, (KPT + QPT) // 16, codes, 0)

    with jax.named_scope("sc_stage"):
        soff = pl.multiple_of(s * KPT, KPT)
        pltpu.sync_copy(cb0_v.at[pl.ds(0, KPT)], shared.at[0, pl.ds(soff, KPT)])
        pltpu.sync_copy(cb1_v.at[pl.ds(0, KPT)], shared.at[1, pl.ds(soff, KPT)])
        plsc.subcore_barrier()
        pltpu.sync_copy(shared.at[0], kc0_v)
        pltpu.sync_copy(shared.at[1], kc1_v)

    UN = 4

    def build(c, _):
        for u in range(UN):
            off = pl.multiple_of(c * 16 * UN + u * 16, 16)
            k0 = kc0_v[pl.ds(off, 16)]
            k1 = kc1_v[pl.ds(off, 16)]
            plsc.store_scatter(cm0, [k0 & MMASK], k0)
            plsc.store_scatter(cm1, [k1 & MMASK], k1)
        return 0

    with jax.named_scope("sc_build"):
        lax.fori_loop(0, L // (16 * UN), build, 0)

    dma_z0.wait()
    dma_z1.wait()
    ones = jnp.full((16,), 1, jnp.int32)

    def contest(c, _):
        for u in range(UN):
            off = pl.multiple_of(c * 16 * UN + u * 16, 16)
            k0 = kc0_v[pl.ds(off, 16)]
            k1 = kc1_v[pl.ds(off, 16)]
            s0 = k0 & MMASK
            s1 = k1 & MMASK
            lost0 = plsc.load_gather(cm0, [s0]) != k0
            lost1 = plsc.load_gather(cm1, [s1]) != k1
            plsc.store_scatter(ct0, [s0], ones, mask=lost0)
            plsc.store_scatter(ct1, [s1], ones, mask=lost1)
        return 0

    with jax.named_scope("sc_contest"):
        lax.fori_loop(0, L // (16 * UN), contest, 0)

    sscope = jax.named_scope("sc_screen")
    sscope.__enter__()
    for v in range(QPT // 16):
        off = KPT + v * 16
        q0 = cb0_v[pl.ds(off, 16)]
        q1 = cb1_v[pl.ds(off, 16)]
        s0 = q0 & MMASK
        s1 = q1 & MMASK
        hit0 = plsc.load_gather(cm0, [s0]) == q0
        hit1 = plsc.load_gather(cm1, [s1]) == q1
        con0 = plsc.load_gather(ct0, [s0]) > 0
        con1 = plsc.load_gather(ct1, [s1]) > 0
        flag = hit0 | hit1
        amb = (~flag) & (con0 | con1)
        flags_v[pl.ds(v * 16, 16)] = flag.astype(jnp.int32)
        n_amb = jnp.sum(amb.astype(jnp.int32))

        @pl.when(n_amb > 0)
        def _verify(v=v, amb=amb):
            def w_cond(carry):
                return jnp.max(carry.astype(jnp.int32)) > 0

            def w_body(carry):
                w = plsc.all_reduce_ffs(carry)
                qloc = jnp.zeros((16,), jnp.int32) + KPT + v * 16 + w
                a0 = plsc.load_gather(cb0_v, [qloc])
                a1 = plsc.load_gather(cb1_v, [qloc])

                def scan(c, acc):
                    for u in range(UN):
                        o = pl.multiple_of(c * 16 * UN + u * 16, 16)
                        acc = (acc | (kc0_v[pl.ds(o, 16)] == a0)
                               | (kc1_v[pl.ds(o, 16)] == a1))
                    return acc

                m = lax.fori_loop(0, L // (16 * UN), scan,
                                  jnp.zeros((16,), jnp.bool_))
                res = (plsc.all_reduce_population_count(m) > 0)
                plsc.store_scatter(flags_v, [qloc - KPT],
                                   res.astype(jnp.int32),
                                   mask=lanes == 0)
                return carry & (lanes != w)

            lax.while_loop(w_cond, w_body, amb)

    sscope.__exit__(None, None, None)
    pltpu.sync_copy(flags_v, out_hbm.at[pl.ds(base, QPT)])


@functools.lru_cache(maxsize=1)
def _make_sc_screen():
    mesh = plsc.VectorSubcoreMesh(core_axis_name="c", subcore_axis_name="s",
                                  num_cores=NCORE, num_subcores=NSUB)
    return pl.kernel(
        _sc_screen_body,
        out_type=jax.ShapeDtypeStruct((B * L,), jnp.int32),
        mesh=mesh,
        compiler_params=pltpu.CompilerParams(needs_layout_passes=False),
        scratch_types=[
            pltpu.VMEM((KPT + QPT, D), jnp.float32),
            pltpu.VMEM((KPT + QPT,), jnp.int32),
            pltpu.VMEM((KPT + QPT,), jnp.int32),
            pltpu.VMEM((L,), jnp.int32),
            pltpu.VMEM((L,), jnp.int32),
            pltpu.VMEM((MAPW,), jnp.int32),
            pltpu.VMEM((MAPW,), jnp.int32),
            pltpu.VMEM((MAPW,), jnp.int32),
            pltpu.VMEM((MAPW,), jnp.int32),
            pltpu.VMEM((QPT,), jnp.int32),
            pltpu.VMEM_SHARED((2, L), jnp.int32),
            pltpu.SemaphoreType.DMA,
            pltpu.SemaphoreType.DMA,
            pltpu.SemaphoreType.DMA,
            pltpu.SemaphoreType.DMA,
        ],
    )


def _pack_weights():
    d = lax.broadcasted_iota(jnp.int32, (G, 2), 0)
    c = lax.broadcasted_iota(jnp.int32, (G, 2), 1)
    p = jnp.left_shift(jnp.int32(1), lax.rem(d, 16)).astype(jnp.float32)
    return jnp.where(c == d // 16, p, 0.0)


def _body(qf_ref, kf_ref, p0_ref, p1_ref, cand_ref, vals_ref):
    qf = qf_ref[0]
    kf = kf_ref[0]
    projs = (p0_ref[...], p1_ref[...])

    W = _pack_weights()

    qcodes, kcodes, qhash, khash = [], [], [], []
    for g in range(2):
        qg = qf[:, g * G:(g + 1) * G]
        kg = kf[:, g * G:(g + 1) * G]
        qb = (qg > 0).astype(jnp.float32)
        kb = (kg > 0).astype(jnp.float32)
        qcodes.append(lax.dot_general(qb, W, (((1,), (0,)), ((), ())),
                                      preferred_element_type=jnp.float32))
        kcodes.append(lax.dot_general(W, kb, (((0,), (1,)), ((), ())),
                                      preferred_element_type=jnp.float32))
        qy = lax.dot_general(qg, projs[g], (((1,), (0,)), ((), ())),
                             preferred_element_type=jnp.float32)
        ky = lax.dot_general(projs[g], kg, (((0,), (1,)), ((), ())),
                             preferred_element_type=jnp.float32)
        qh = jnp.floor(qy / BW)
        kh = jnp.floor(ky / BW)
        qhash.append(qh - jnp.floor(qh / NB) * NB)
        khash.append(kh - jnp.floor(kh / NB) * NB)

    code_eq = []
    for g in range(2):
        eq = ((qcodes[g][:, 0:1] == kcodes[g][0:1, :]) &
              (qcodes[g][:, 1:2] == kcodes[g][1:2, :]))
        code_eq.append(eq)
    screen = code_eq[0] | code_eq[1]
    any_match = jnp.sum(screen.astype(jnp.int32)) > 0

    cand_ref[...] = jnp.full((1, BQ, K), -1, dtype=jnp.int32)
    vals_ref[...] = jnp.full((1, BQ, K), NEG, dtype=jnp.float32)

    @pl.when(any_match)
    def _heavy():
        full_mask = jnp.zeros((BQ, L), dtype=jnp.bool_)
        for g in range(2):
            lsh = jnp.zeros((BQ, L), dtype=jnp.bool_)
            for h in range(NH):
                lsh = lsh | (qhash[g][:, h:h + 1] == khash[g][h:h + 1, :])
            full_mask = full_mask | (code_eq[g] & lsh)
        scores = lax.dot_general(qf, kf, (((1,), (1,)), ((), ())),
                                 preferred_element_type=jnp.float32)
        masked = jnp.where(full_mask, scores, NEG)
        iota_k = lax.broadcasted_iota(jnp.int32, (BQ, L), 1)
        iota_c = lax.broadcasted_iota(jnp.int32, (BQ, K), 1)

        def step(j, carry):
            m_vals, out_v, out_i = carry
            mx = jnp.max(m_vals, axis=1, keepdims=True)
            idx = jnp.min(jnp.where(m_vals == mx, iota_k, L),
                          axis=1, keepdims=True)
            col = iota_c == j
            out_v = jnp.where(col, mx, out_v)
            out_i = jnp.where(col, jnp.where(mx > NEG, idx, -1), out_i)
            return jnp.where(iota_k == idx, NEG, m_vals), out_v, out_i

        _, out_v, out_i = lax.fori_loop(
            0, K, step,
            (masked,
             jnp.full((BQ, K), NEG, dtype=jnp.float32),
             jnp.full((BQ, K), -1, dtype=jnp.int32)))
        vals_ref[0] = out_v
        cand_ref[0] = out_i


def _run(qf, kf, p0, p1):
    qt = L // BQ
    grid = (B, qt)
    return pl.pallas_call(
        _body,
        grid=grid,
        in_specs=[
            pl.BlockSpec((1, BQ, D), lambda b, t: (b, t, 0)),
            pl.BlockSpec((1, L, D), lambda b, t: (b, 0, 0)),
            pl.BlockSpec((G, NH), lambda b, t: (0, 0)),
            pl.BlockSpec((G, NH), lambda b, t: (0, 0)),
        ],
        out_specs=[
            pl.BlockSpec((1, BQ, K), lambda b, t: (b, t, 0)),
            pl.BlockSpec((1, BQ, K), lambda b, t: (b, t, 0)),
        ],
        out_shape=[
            jax.ShapeDtypeStruct((B, L, K), jnp.int32),
            jax.ShapeDtypeStruct((B, L, K), jnp.float32),
        ],
    )(qf, kf, p0, p1)


@jax.jit
def _dispatch(qf, kf, p0, p1):
    flags = _make_sc_screen()(qf.reshape(B * L, D), kf.reshape(B * L, D),
                              jnp.zeros((MAPW,), jnp.int32))
    any_match = jnp.sum(flags) > 0
    return lax.cond(
        any_match,
        lambda: _run(qf, kf, p0, p1),
        lambda: (jnp.full((B, L, K), -1, dtype=jnp.int32),
                 jnp.full((B, L, K), NEG, dtype=jnp.float32)))


def kernel(query_features, key_features, head_idx, lsh_proj_g0, lsh_proj_g1):
    cand, vals = _dispatch(query_features, key_features,
                           lsh_proj_g0, lsh_proj_g1)
    return cand, vals


# speedup vs baseline: 1.4424x; 1.1968x over previous
import functools

import jax
import jax.numpy as jnp
from jax import lax
from jax.experimental import pallas as pl
from jax.experimental.pallas import tpu as pltpu
from jax.experimental.pallas import tpu_sc as plsc

B, L, D = 2, 2048, 64
G = 32
NH = 4
BW = 4.0
NB = 64
K = 64
BQ = 256
NEG = float(jnp.finfo(jnp.float32).min)

NCORE, NSUB = 2, 16
TPB = NSUB
KPT = L // TPB
QPT = L // TPB
MAPB = 14
MAPW = 1 << MAPB
MMASK = MAPW - 1


def _sc_screen_body(qf_hbm, kf_hbm, out_hbm,
                    feat_v, cb0_v, cb1_v, kc0_v, kc1_v,
                    cm0, cm1, ct0, ct1, flags_v, shared, sem, sem2):
    b = lax.axis_index("c")
    s = lax.axis_index("s")
    base = pl.multiple_of(s * KPT, KPT)

    dma_k = pltpu.make_async_copy(kf_hbm.at[b, pl.ds(base, KPT)],
                                  feat_v.at[pl.ds(0, KPT)], sem)
    dma_q = pltpu.make_async_copy(qf_hbm.at[b, pl.ds(base, QPT)],
                                  feat_v.at[pl.ds(KPT, QPT)], sem2)
    dma_k.start()
    dma_q.start()

    zeros16 = jnp.zeros((16,), jnp.int32)

    def zero_maps(c, _):
        for u in range(8):
            off = pl.multiple_of(c * 128 + u * 16, 16)
            ct0[pl.ds(off, 16)] = zeros16
            ct1[pl.ds(off, 16)] = zeros16
        return 0

    with jax.named_scope("sc_zero"):
        lax.fori_loop(0, MAPW // 128, zero_maps, 0)

    lanes = lax.iota(jnp.int32, 16)
    dma_k.wait()
    dma_q.wait()

    def codes(i, _):
        rows = lanes + i * 16
        acc0 = jnp.zeros((16,), jnp.int32)
        acc1 = jnp.zeros((16,), jnp.int32)
        for d in range(G):
            sh = (lanes + d) & (G - 1)
            v0 = plsc.load_gather(feat_v, [rows, sh])
            v1 = plsc.load_gather(feat_v, [rows, sh + G])
            acc0 = acc0 | ((v0 > 0).astype(jnp.int32) << sh)
            acc1 = acc1 | ((v1 > 0).astype(jnp.int32) << sh)
        off = pl.multiple_of(i * 16, 16)
        cb0_v[pl.ds(off, 16)] = acc0
        cb1_v[pl.ds(off, 16)] = acc1
        return 0

    with jax.named_scope("sc_codes"):
        lax.fori_loop(0, (KPT + QPT) // 16, codes, 0)

    with jax.named_scope("sc_stage"):
        soff = pl.multiple_of(s * KPT, KPT)
        pltpu.sync_copy(cb0_v.at[pl.ds(0, KPT)], shared.at[0, pl.ds(soff, KPT)])
        pltpu.sync_copy(cb1_v.at[pl.ds(0, KPT)], shared.at[1, pl.ds(soff, KPT)])
        plsc.subcore_barrier()
        pltpu.sync_copy(shared.at[0], kc0_v)
        pltpu.sync_copy(shared.at[1], kc1_v)

    UN = 4

    def build(c, _):
        for u in range(UN):
            off = pl.multiple_of(c * 16 * UN + u * 16, 16)
            k0 = kc0_v[pl.ds(off, 16)]
            k1 = kc1_v[pl.ds(off, 16)]
            plsc.store_scatter(cm0, [k0 & MMASK], k0)
            plsc.store_scatter(cm1, [k1 & MMASK], k1)
        return 0

    with jax.named_scope("sc_build"):
        lax.fori_loop(0, L // (16 * UN), build, 0)

    ones = jnp.full((16,), 1, jnp.int32)

    def contest(c, _):
        for u in range(UN):
            off = pl.multiple_of(c * 16 * UN + u * 16, 16)
            k0 = kc0_v[pl.ds(off, 16)]
            k1 = kc1_v[pl.ds(off, 16)]
            s0 = k0 & MMASK
            s1 = k1 & MMASK
            lost0 = plsc.load_gather(cm0, [s0]) != k0
            lost1 = plsc.load_gather(cm1, [s1]) != k1
            plsc.store_scatter(ct0, [s0], ones, mask=lost0)
            plsc.store_scatter(ct1, [s1], ones, mask=lost1)
        return 0

    with jax.named_scope("sc_contest"):
        lax.fori_loop(0, L // (16 * UN), contest, 0)

    sscope = jax.named_scope("sc_screen")
    sscope.__enter__()
    for v in range(QPT // 16):
        off = KPT + v * 16
        q0 = cb0_v[pl.ds(off, 16)]
        q1 = cb1_v[pl.ds(off, 16)]
        s0 = q0 & MMASK
        s1 = q1 & MMASK
        hit0 = plsc.load_gather(cm0, [s0]) == q0
        hit1 = plsc.load_gather(cm1, [s1]) == q1
        con0 = plsc.load_gather(ct0, [s0]) > 0
        con1 = plsc.load_gather(ct1, [s1]) > 0
        flag = hit0 | hit1
        amb = (~flag) & (con0 | con1)
        flags_v[pl.ds(v * 16, 16)] = flag.astype(jnp.int32)
        n_amb = jnp.sum(amb.astype(jnp.int32))

        @pl.when(n_amb > 0)
        def _verify(v=v, amb=amb):
            def w_cond(carry):
                return jnp.max(carry.astype(jnp.int32)) > 0

            def w_body(carry):
                w = plsc.all_reduce_ffs(carry)
                qloc = jnp.zeros((16,), jnp.int32) + KPT + v * 16 + w
                a0 = plsc.load_gather(cb0_v, [qloc])
                a1 = plsc.load_gather(cb1_v, [qloc])

                def scan(c, acc):
                    for u in range(UN):
                        o = pl.multiple_of(c * 16 * UN + u * 16, 16)
                        acc = (acc | (kc0_v[pl.ds(o, 16)] == a0)
                               | (kc1_v[pl.ds(o, 16)] == a1))
                    return acc

                m = lax.fori_loop(0, L // (16 * UN), scan,
                                  jnp.zeros((16,), jnp.bool_))
                res = (plsc.all_reduce_population_count(m) > 0)
                plsc.store_scatter(flags_v, [qloc - KPT],
                                   res.astype(jnp.int32),
                                   mask=lanes == 0)
                return carry & (lanes != w)

            lax.while_loop(w_cond, w_body, amb)

    sscope.__exit__(None, None, None)
    anyv = jnp.zeros((16,), jnp.int32)
    for v in range(QPT // 16):
        anyv = anyv | flags_v[pl.ds(pl.multiple_of(v * 16, 16), 16)]
    flags_v[pl.ds(0, 16)] = anyv
    w = pl.multiple_of((b * NSUB + s) * 16, 16)
    pltpu.sync_copy(flags_v.at[pl.ds(0, 16)], out_hbm.at[pl.ds(w, 16)])


@functools.lru_cache(maxsize=1)
def _make_sc_screen():
    mesh = plsc.VectorSubcoreMesh(core_axis_name="c", subcore_axis_name="s",
                                  num_cores=NCORE, num_subcores=NSUB)
    return pl.kernel(
        _sc_screen_body,
        out_type=jax.ShapeDtypeStruct((NCORE * NSUB * 16,), jnp.int32),
        mesh=mesh,
        compiler_params=pltpu.CompilerParams(needs_layout_passes=False),
        scratch_types=[
            pltpu.VMEM((KPT + QPT, D), jnp.float32),
            pltpu.VMEM((KPT + QPT,), jnp.int32),
            pltpu.VMEM((KPT + QPT,), jnp.int32),
            pltpu.VMEM((L,), jnp.int32),
            pltpu.VMEM((L,), jnp.int32),
            pltpu.VMEM((MAPW,), jnp.int32),
            pltpu.VMEM((MAPW,), jnp.int32),
            pltpu.VMEM((MAPW,), jnp.int32),
            pltpu.VMEM((MAPW,), jnp.int32),
            pltpu.VMEM((QPT,), jnp.int32),
            pltpu.VMEM_SHARED((2, L), jnp.int32),
            pltpu.SemaphoreType.DMA,
            pltpu.SemaphoreType.DMA,
        ],
    )


def _pack_weights():
    d = lax.broadcasted_iota(jnp.int32, (G, 2), 0)
    c = lax.broadcasted_iota(jnp.int32, (G, 2), 1)
    p = jnp.left_shift(jnp.int32(1), lax.rem(d, 16)).astype(jnp.float32)
    return jnp.where(c == d // 16, p, 0.0)


def _body(qf_ref, kf_ref, p0_ref, p1_ref, cand_ref, vals_ref):
    qf = qf_ref[0]
    kf = kf_ref[0]
    projs = (p0_ref[...], p1_ref[...])

    W = _pack_weights()

    qcodes, kcodes, qhash, khash = [], [], [], []
    for g in range(2):
        qg = qf[:, g * G:(g + 1) * G]
        kg = kf[:, g * G:(g + 1) * G]
        qb = (qg > 0).astype(jnp.float32)
        kb = (kg > 0).astype(jnp.float32)
        qcodes.append(lax.dot_general(qb, W, (((1,), (0,)), ((), ())),
                                      preferred_element_type=jnp.float32))
        kcodes.append(lax.dot_general(W, kb, (((0,), (1,)), ((), ())),
                                      preferred_element_type=jnp.float32))
        qy = lax.dot_general(qg, projs[g], (((1,), (0,)), ((), ())),
                             preferred_element_type=jnp.float32)
        ky = lax.dot_general(projs[g], kg, (((0,), (1,)), ((), ())),
                             preferred_element_type=jnp.float32)
        qh = jnp.floor(qy / BW)
        kh = jnp.floor(ky / BW)
        qhash.append(qh - jnp.floor(qh / NB) * NB)
        khash.append(kh - jnp.floor(kh / NB) * NB)

    code_eq = []
    for g in range(2):
        eq = ((qcodes[g][:, 0:1] == kcodes[g][0:1, :]) &
              (qcodes[g][:, 1:2] == kcodes[g][1:2, :]))
        code_eq.append(eq)
    screen = code_eq[0] | code_eq[1]
    any_match = jnp.sum(screen.astype(jnp.int32)) > 0

    cand_ref[...] = jnp.full((1, BQ, K), -1, dtype=jnp.int32)
    vals_ref[...] = jnp.full((1, BQ, K), NEG, dtype=jnp.float32)

    @pl.when(any_match)
    def _heavy():
        full_mask = jnp.zeros((BQ, L), dtype=jnp.bool_)
        for g in range(2):
            lsh = jnp.zeros((BQ, L), dtype=jnp.bool_)
            for h in range(NH):
                lsh = lsh | (qhash[g][:, h:h + 1] == khash[g][h:h + 1, :])
            full_mask = full_mask | (code_eq[g] & lsh)
        scores = lax.dot_general(qf, kf, (((1,), (1,)), ((), ())),
                                 preferred_element_type=jnp.float32)
        masked = jnp.where(full_mask, scores, NEG)
        iota_k = lax.broadcasted_iota(jnp.int32, (BQ, L), 1)
        iota_c = lax.broadcasted_iota(jnp.int32, (BQ, K), 1)

        def step(j, carry):
            m_vals, out_v, out_i = carry
            mx = jnp.max(m_vals, axis=1, keepdims=True)
            idx = jnp.min(jnp.where(m_vals == mx, iota_k, L),
                          axis=1, keepdims=True)
            col = iota_c == j
            out_v = jnp.where(col, mx, out_v)
            out_i = jnp.where(col, jnp.where(mx > NEG, idx, -1), out_i)
            return jnp.where(iota_k == idx, NEG, m_vals), out_v, out_i

        _, out_v, out_i = lax.fori_loop(
            0, K, step,
            (masked,
             jnp.full((BQ, K), NEG, dtype=jnp.float32),
             jnp.full((BQ, K), -1, dtype=jnp.int32)))
        vals_ref[0] = out_v
        cand_ref[0] = out_i


def _run(qf, kf, p0, p1):
    qt = L // BQ
    grid = (B, qt)
    return pl.pallas_call(
        _body,
        grid=grid,
        in_specs=[
            pl.BlockSpec((1, BQ, D), lambda b, t: (b, t, 0)),
            pl.BlockSpec((1, L, D), lambda b, t: (b, 0, 0)),
            pl.BlockSpec((G, NH), lambda b, t: (0, 0)),
            pl.BlockSpec((G, NH), lambda b, t: (0, 0)),
        ],
        out_specs=[
            pl.BlockSpec((1, BQ, K), lambda b, t: (b, t, 0)),
            pl.BlockSpec((1, BQ, K), lambda b, t: (b, t, 0)),
        ],
        out_shape=[
            jax.ShapeDtypeStruct((B, L, K), jnp.int32),
            jax.ShapeDtypeStruct((B, L, K), jnp.float32),
        ],
    )(qf, kf, p0, p1)


@jax.jit
def _dispatch(qf, kf, p0, p1):
    flags = _make_sc_screen()(qf, kf)
    any_match = jnp.sum(flags) > 0
    return lax.cond(
        any_match,
        lambda: _run(qf, kf, p0, p1),
        lambda: (jnp.full((B, L, K), -1, dtype=jnp.int32),
                 jnp.full((B, L, K), NEG, dtype=jnp.float32)))


def kernel(query_features, key_features, head_idx, lsh_proj_g0, lsh_proj_g1):
    cand, vals = _dispatch(query_features, key_features,
                           lsh_proj_g0, lsh_proj_g1)
    return cand, vals


# speedup vs baseline: 1.6221x; 1.1245x over previous
import functools

import jax
import jax.numpy as jnp
from jax import lax
from jax.experimental import pallas as pl
from jax.experimental.pallas import tpu as pltpu
from jax.experimental.pallas import tpu_sc as plsc

B, L, D = 2, 2048, 64
G = 32
NH = 4
BW = 4.0
NB = 64
K = 64
BQ = 256
NEG = float(jnp.finfo(jnp.float32).min)

NCORE, NSUB = 2, 16
TPB = NSUB
KPT = L // TPB
QPT = L // TPB
MAPB = 14
MAPW = 1 << MAPB
MMASK = MAPW - 1


def _sc_screen_body(qc0_hbm, qc1_hbm, kc0_hbm, kc1_hbm, out_hbm,
                    kc0_v, kc1_v, qc0_v, qc1_v,
                    cm0, cm1, ct0, ct1, flags_v, sem, sem2, sem3, sem4):
    b = lax.axis_index("c")
    s = lax.axis_index("s")
    base = pl.multiple_of(s * QPT, QPT)
    lanes = lax.iota(jnp.int32, 16)

    dmas = [
        pltpu.make_async_copy(kc0_hbm.at[b, 0], kc0_v, sem),
        pltpu.make_async_copy(kc1_hbm.at[b, 0], kc1_v, sem2),
        pltpu.make_async_copy(qc0_hbm.at[b, 0, pl.ds(base, QPT)], qc0_v, sem3),
        pltpu.make_async_copy(qc1_hbm.at[b, 0, pl.ds(base, QPT)], qc1_v, sem4),
    ]
    for dma in dmas:
        dma.start()

    zeros16 = jnp.zeros((16,), jnp.int32)

    def zero_maps(c, _):
        for u in range(8):
            off = pl.multiple_of(c * 128 + u * 16, 16)
            ct0[pl.ds(off, 16)] = zeros16
            ct1[pl.ds(off, 16)] = zeros16
        return 0

    with jax.named_scope("sc_zero"):
        lax.fori_loop(0, MAPW // 128, zero_maps, 0)
    for dma in dmas:
        dma.wait()

    UN = 4

    def build(c, _):
        for u in range(UN):
            off = pl.multiple_of(c * 16 * UN + u * 16, 16)
            k0 = kc0_v[pl.ds(off, 16)]
            k1 = kc1_v[pl.ds(off, 16)]
            plsc.store_scatter(cm0, [k0 & MMASK], k0)
            plsc.store_scatter(cm1, [k1 & MMASK], k1)
        return 0

    with jax.named_scope("sc_build"):
        lax.fori_loop(0, L // (16 * UN), build, 0)

    ones = jnp.full((16,), 1, jnp.int32)

    def contest(c, _):
        for u in range(UN):
            off = pl.multiple_of(c * 16 * UN + u * 16, 16)
            k0 = kc0_v[pl.ds(off, 16)]
            k1 = kc1_v[pl.ds(off, 16)]
            s0 = k0 & MMASK
            s1 = k1 & MMASK
            lost0 = plsc.load_gather(cm0, [s0]) != k0
            lost1 = plsc.load_gather(cm1, [s1]) != k1
            plsc.store_scatter(ct0, [s0], ones, mask=lost0)
            plsc.store_scatter(ct1, [s1], ones, mask=lost1)
        return 0

    with jax.named_scope("sc_contest"):
        lax.fori_loop(0, L // (16 * UN), contest, 0)

    sscope = jax.named_scope("sc_screen")
    sscope.__enter__()
    for v in range(QPT // 16):
        off = v * 16
        q0 = qc0_v[pl.ds(off, 16)]
        q1 = qc1_v[pl.ds(off, 16)]
        s0 = q0 & MMASK
        s1 = q1 & MMASK
        hit0 = plsc.load_gather(cm0, [s0]) == q0
        hit1 = plsc.load_gather(cm1, [s1]) == q1
        con0 = plsc.load_gather(ct0, [s0]) > 0
        con1 = plsc.load_gather(ct1, [s1]) > 0
        flag = hit0 | hit1
        amb = (~flag) & (con0 | con1)
        flags_v[pl.ds(off, 16)] = flag.astype(jnp.int32)
        n_amb = jnp.sum(amb.astype(jnp.int32))

        @pl.when(n_amb > 0)
        def _verify(v=v, amb=amb):
            def w_cond(carry):
                return jnp.max(carry.astype(jnp.int32)) > 0

            def w_body(carry):
                w = plsc.all_reduce_ffs(carry)
                qloc = jnp.zeros((16,), jnp.int32) + v * 16 + w
                a0 = plsc.load_gather(qc0_v, [qloc])
                a1 = plsc.load_gather(qc1_v, [qloc])

                def scan(c, acc):
                    for u in range(UN):
                        o = pl.multiple_of(c * 16 * UN + u * 16, 16)
                        acc = (acc | (kc0_v[pl.ds(o, 16)] == a0)
                               | (kc1_v[pl.ds(o, 16)] == a1))
                    return acc

                m = lax.fori_loop(0, L // (16 * UN), scan,
                                  jnp.zeros((16,), jnp.bool_))
                res = (plsc.all_reduce_population_count(m) > 0)
                plsc.store_scatter(flags_v, [qloc],
                                   res.astype(jnp.int32),
                                   mask=lanes == 0)
                return carry & (lanes != w)

            lax.while_loop(w_cond, w_body, amb)

    sscope.__exit__(None, None, None)
    anyv = jnp.zeros((16,), jnp.int32)
    for v in range(QPT // 16):
        anyv = anyv | flags_v[pl.ds(pl.multiple_of(v * 16, 16), 16)]
    flags_v[pl.ds(0, 16)] = anyv
    w = pl.multiple_of((b * NSUB + s) * 16, 16)
    pltpu.sync_copy(flags_v.at[pl.ds(0, 16)], out_hbm.at[pl.ds(w, 16)])


@functools.lru_cache(maxsize=1)
def _make_sc_screen():
    mesh = plsc.VectorSubcoreMesh(core_axis_name="c", subcore_axis_name="s",
                                  num_cores=NCORE, num_subcores=NSUB)
    return pl.kernel(
        _sc_screen_body,
        out_type=jax.ShapeDtypeStruct((NCORE * NSUB * 16,), jnp.int32),
        mesh=mesh,
        compiler_params=pltpu.CompilerParams(needs_layout_passes=False),
        scratch_types=[
            pltpu.VMEM((L,), jnp.int32),
            pltpu.VMEM((L,), jnp.int32),
            pltpu.VMEM((QPT,), jnp.int32),
            pltpu.VMEM((QPT,), jnp.int32),
            pltpu.VMEM((MAPW,), jnp.int32),
            pltpu.VMEM((MAPW,), jnp.int32),
            pltpu.VMEM((MAPW,), jnp.int32),
            pltpu.VMEM((MAPW,), jnp.int32),
            pltpu.VMEM((QPT,), jnp.int32),
            pltpu.SemaphoreType.DMA,
            pltpu.SemaphoreType.DMA,
            pltpu.SemaphoreType.DMA,
            pltpu.SemaphoreType.DMA,
        ],
    )


def _pack_weights():
    d = lax.broadcasted_iota(jnp.int32, (G, 2), 0)
    c = lax.broadcasted_iota(jnp.int32, (G, 2), 1)
    p = jnp.left_shift(jnp.int32(1), lax.rem(d, 16)).astype(jnp.float32)
    return jnp.where(c == d // 16, p, 0.0)


def _body(qf_ref, kf_ref, p0_ref, p1_ref, cand_ref, vals_ref):
    qf = qf_ref[0]
    kf = kf_ref[0]
    projs = (p0_ref[...], p1_ref[...])

    W = _pack_weights()

    qcodes, kcodes, qhash, khash = [], [], [], []
    for g in range(2):
        qg = qf[:, g * G:(g + 1) * G]
        kg = kf[:, g * G:(g + 1) * G]
        qb = (qg > 0).astype(jnp.float32)
        kb = (kg > 0).astype(jnp.float32)
        qcodes.append(lax.dot_general(qb, W, (((1,), (0,)), ((), ())),
                                      preferred_element_type=jnp.float32))
        kcodes.append(lax.dot_general(W, kb, (((0,), (1,)), ((), ())),
                                      preferred_element_type=jnp.float32))
        qy = lax.dot_general(qg, projs[g], (((1,), (0,)), ((), ())),
                             preferred_element_type=jnp.float32)
        ky = lax.dot_general(projs[g], kg, (((0,), (1,)), ((), ())),
                             preferred_element_type=jnp.float32)
        qh = jnp.floor(qy / BW)
        kh = jnp.floor(ky / BW)
        qhash.append(qh - jnp.floor(qh / NB) * NB)
        khash.append(kh - jnp.floor(kh / NB) * NB)

    code_eq = []
    for g in range(2):
        eq = ((qcodes[g][:, 0:1] == kcodes[g][0:1, :]) &
              (qcodes[g][:, 1:2] == kcodes[g][1:2, :]))
        code_eq.append(eq)
    screen = code_eq[0] | code_eq[1]
    any_match = jnp.sum(screen.astype(jnp.int32)) > 0

    cand_ref[...] = jnp.full((1, BQ, K), -1, dtype=jnp.int32)
    vals_ref[...] = jnp.full((1, BQ, K), NEG, dtype=jnp.float32)

    @pl.when(any_match)
    def _heavy():
        full_mask = jnp.zeros((BQ, L), dtype=jnp.bool_)
        for g in range(2):
            lsh = jnp.zeros((BQ, L), dtype=jnp.bool_)
            for h in range(NH):
                lsh = lsh | (qhash[g][:, h:h + 1] == khash[g][h:h + 1, :])
            full_mask = full_mask | (code_eq[g] & lsh)
        scores = lax.dot_general(qf, kf, (((1,), (1,)), ((), ())),
                                 preferred_element_type=jnp.float32)
        masked = jnp.where(full_mask, scores, NEG)
        iota_k = lax.broadcasted_iota(jnp.int32, (BQ, L), 1)
        iota_c = lax.broadcasted_iota(jnp.int32, (BQ, K), 1)

        def step(j, carry):
            m_vals, out_v, out_i = carry
            mx = jnp.max(m_vals, axis=1, keepdims=True)
            idx = jnp.min(jnp.where(m_vals == mx, iota_k, L),
                          axis=1, keepdims=True)
            col = iota_c == j
            out_v = jnp.where(col, mx, out_v)
            out_i = jnp.where(col, jnp.where(mx > NEG, idx, -1), out_i)
            return jnp.where(iota_k == idx, NEG, m_vals), out_v, out_i

        _, out_v, out_i = lax.fori_loop(
            0, K, step,
            (masked,
             jnp.full((BQ, K), NEG, dtype=jnp.float32),
             jnp.full((BQ, K), -1, dtype=jnp.int32)))
        vals_ref[0] = out_v
        cand_ref[0] = out_i


def _run(qf, kf, p0, p1):
    qt = L // BQ
    grid = (B, qt)
    return pl.pallas_call(
        _body,
        grid=grid,
        in_specs=[
            pl.BlockSpec((1, BQ, D), lambda b, t: (b, t, 0)),
            pl.BlockSpec((1, L, D), lambda b, t: (b, 0, 0)),
            pl.BlockSpec((G, NH), lambda b, t: (0, 0)),
            pl.BlockSpec((G, NH), lambda b, t: (0, 0)),
        ],
        out_specs=[
            pl.BlockSpec((1, BQ, K), lambda b, t: (b, t, 0)),
            pl.BlockSpec((1, BQ, K), lambda b, t: (b, t, 0)),
        ],
        out_shape=[
            jax.ShapeDtypeStruct((B, L, K), jnp.int32),
            jax.ShapeDtypeStruct((B, L, K), jnp.float32),
        ],
    )(qf, kf, p0, p1)


def _codes_body(qf_ref, kf_ref, qc0_ref, qc1_ref, kc0_ref, kc1_ref):
    W = _pack_weights()
    for feat_ref, (c0_ref, c1_ref) in ((qf_ref, (qc0_ref, qc1_ref)),
                                       (kf_ref, (kc0_ref, kc1_ref))):
        x = feat_ref[0]
        for g, c_ref in enumerate((c0_ref, c1_ref)):
            xb = (x[:, g * G:(g + 1) * G] > 0).astype(jnp.float32)
            halves = lax.dot_general(W, xb, (((0,), (1,)), ((), ())),
                                     preferred_element_type=jnp.float32)
            code = (halves[0:1, :].astype(jnp.int32) +
                    (halves[1:2, :].astype(jnp.int32) << 16))
            c_ref[0] = code


def _codes(qf, kf):
    return pl.pallas_call(
        _codes_body,
        grid=(B,),
        in_specs=[
            pl.BlockSpec((1, L, D), lambda b: (b, 0, 0)),
            pl.BlockSpec((1, L, D), lambda b: (b, 0, 0)),
        ],
        out_specs=[pl.BlockSpec((1, 1, L), lambda b: (b, 0, 0))] * 4,
        out_shape=[jax.ShapeDtypeStruct((B, 1, L), jnp.int32)] * 4,
    )(qf, kf)


@jax.jit
def _dispatch(qf, kf, p0, p1):
    qc0, qc1, kc0, kc1 = _codes(qf, kf)
    flags = _make_sc_screen()(qc0, qc1, kc0, kc1)
    any_match = jnp.sum(flags) > 0
    return lax.cond(
        any_match,
        lambda: _run(qf, kf, p0, p1),
        lambda: (jnp.full((B, L, K), -1, dtype=jnp.int32),
                 jnp.full((B, L, K), NEG, dtype=jnp.float32)))


def kernel(query_features, key_features, head_idx, lsh_proj_g0, lsh_proj_g1):
    cand, vals = _dispatch(query_features, key_features,
                           lsh_proj_g0, lsh_proj_g1)
    return cand, vals


# speedup vs baseline: 1.6407x; 1.0115x over previous
import functools

import jax
import jax.numpy as jnp
from jax import lax
from jax.experimental import pallas as pl
from jax.experimental.pallas import tpu as pltpu
from jax.experimental.pallas import tpu_sc as plsc

B, L, D = 2, 2048, 64
G = 32
NH = 4
BW = 4.0
NB = 64
K = 64
BQ = 256
NEG = float(jnp.finfo(jnp.float32).min)

NCORE, NSUB = 2, 16
TPB = NSUB
KPT = L // TPB
QPT = L // TPB
MAPB = 14
MAPW = 1 << MAPB
MMASK = MAPW - 1


def _sc_screen_body(qc0_hbm, qc1_hbm, kc0_hbm, kc1_hbm, out_hbm,
                    kc0_v, kc1_v, qc0_v, qc1_v,
                    cm0, cm1, ct0, ct1, flags_v, sem, sem2, sem3, sem4):
    b = lax.axis_index("c")
    s = lax.axis_index("s")
    base = pl.multiple_of(s * QPT, QPT)
    lanes = lax.iota(jnp.int32, 16)

    dmas = [
        pltpu.make_async_copy(kc0_hbm.at[b, 0], kc0_v, sem),
        pltpu.make_async_copy(kc1_hbm.at[b, 0], kc1_v, sem2),
        pltpu.make_async_copy(qc0_hbm.at[b, 0, pl.ds(base, QPT)], qc0_v, sem3),
        pltpu.make_async_copy(qc1_hbm.at[b, 0, pl.ds(base, QPT)], qc1_v, sem4),
    ]
    for dma in dmas:
        dma.start()

    zeros16 = jnp.zeros((16,), jnp.int32)

    def zero_maps(c, _):
        for u in range(8):
            off = pl.multiple_of(c * 128 + u * 16, 16)
            ct0[pl.ds(off, 16)] = zeros16
            ct1[pl.ds(off, 16)] = zeros16
        return 0

    with jax.named_scope("sc_zero"):
        lax.fori_loop(0, MAPW // 128, zero_maps, 0)
    for dma in dmas:
        dma.wait()

    UN = 4

    def build(c, _):
        for u in range(UN):
            off = pl.multiple_of(c * 16 * UN + u * 16, 16)
            k0 = kc0_v[pl.ds(off, 16)]
            k1 = kc1_v[pl.ds(off, 16)]
            plsc.store_scatter(cm0, [k0 & MMASK], k0)
            plsc.store_scatter(cm1, [k1 & MMASK], k1)
        return 0

    with jax.named_scope("sc_build"):
        lax.fori_loop(0, L // (16 * UN), build, 0)

    ones = jnp.full((16,), 1, jnp.int32)

    def contest(c, _):
        for u in range(UN):
            off = pl.multiple_of(c * 16 * UN + u * 16, 16)
            k0 = kc0_v[pl.ds(off, 16)]
            k1 = kc1_v[pl.ds(off, 16)]
            s0 = k0 & MMASK
            s1 = k1 & MMASK
            lost0 = plsc.load_gather(cm0, [s0]) != k0
            lost1 = plsc.load_gather(cm1, [s1]) != k1
            plsc.store_scatter(ct0, [s0], ones, mask=lost0)
            plsc.store_scatter(ct1, [s1], ones, mask=lost1)
        return 0

    with jax.named_scope("sc_contest"):
        lax.fori_loop(0, L // (16 * UN), contest, 0)

    sscope = jax.named_scope("sc_screen")
    sscope.__enter__()

    def screen(v, _):
        off = pl.multiple_of(v * 16, 16)
        q0 = qc0_v[pl.ds(off, 16)]
        q1 = qc1_v[pl.ds(off, 16)]
        s0 = q0 & MMASK
        s1 = q1 & MMASK
        hit0 = plsc.load_gather(cm0, [s0]) == q0
        hit1 = plsc.load_gather(cm1, [s1]) == q1
        con0 = plsc.load_gather(ct0, [s0]) > 0
        con1 = plsc.load_gather(ct1, [s1]) > 0
        flag = hit0 | hit1
        amb = (~flag) & (con0 | con1)
        flags_v[pl.ds(off, 16)] = flag.astype(jnp.int32)
        n_amb = jnp.sum(amb.astype(jnp.int32))

        @pl.when(n_amb > 0)
        def _verify():
            def w_cond(carry):
                return jnp.max(carry.astype(jnp.int32)) > 0

            def w_body(carry):
                w = plsc.all_reduce_ffs(carry)
                qloc = jnp.zeros((16,), jnp.int32) + v * 16 + w
                a0 = plsc.load_gather(qc0_v, [qloc])
                a1 = plsc.load_gather(qc1_v, [qloc])

                def scan(c, acc):
                    for u in range(UN):
                        o = pl.multiple_of(c * 16 * UN + u * 16, 16)
                        acc = (acc | (kc0_v[pl.ds(o, 16)] == a0)
                               | (kc1_v[pl.ds(o, 16)] == a1))
                    return acc

                m = lax.fori_loop(0, L // (16 * UN), scan,
                                  jnp.zeros((16,), jnp.bool_))
                res = (plsc.all_reduce_population_count(m) > 0)
                plsc.store_scatter(flags_v, [qloc],
                                   res.astype(jnp.int32),
                                   mask=lanes == 0)
                return carry & (lanes != w)

            lax.while_loop(w_cond, w_body, amb)

        return 0

    lax.fori_loop(0, QPT // 16, screen, 0)

    sscope.__exit__(None, None, None)

    def orred(v, acc):
        return acc | flags_v[pl.ds(pl.multiple_of(v * 16, 16), 16)]

    anyv = lax.fori_loop(0, QPT // 16, orred, jnp.zeros((16,), jnp.int32))
    flags_v[pl.ds(0, 16)] = anyv
    w = pl.multiple_of((b * NSUB + s) * 16, 16)
    pltpu.sync_copy(flags_v.at[pl.ds(0, 16)], out_hbm.at[pl.ds(w, 16)])


@functools.lru_cache(maxsize=1)
def _make_sc_screen():
    mesh = plsc.VectorSubcoreMesh(core_axis_name="c", subcore_axis_name="s",
                                  num_cores=NCORE, num_subcores=NSUB)
    return pl.kernel(
        _sc_screen_body,
        out_type=jax.ShapeDtypeStruct((NCORE * NSUB * 16,), jnp.int32),
        mesh=mesh,
        compiler_params=pltpu.CompilerParams(needs_layout_passes=False),
        scratch_types=[
            pltpu.VMEM((L,), jnp.int32),
            pltpu.VMEM((L,), jnp.int32),
            pltpu.VMEM((QPT,), jnp.int32),
            pltpu.VMEM((QPT,), jnp.int32),
            pltpu.VMEM((MAPW,), jnp.int32),
            pltpu.VMEM((MAPW,), jnp.int32),
            pltpu.VMEM((MAPW,), jnp.int32),
            pltpu.VMEM((MAPW,), jnp.int32),
            pltpu.VMEM((QPT,), jnp.int32),
            pltpu.SemaphoreType.DMA,
            pltpu.SemaphoreType.DMA,
            pltpu.SemaphoreType.DMA,
            pltpu.SemaphoreType.DMA,
        ],
    )


def _pack_weights():
    d = lax.broadcasted_iota(jnp.int32, (G, 2), 0)
    c = lax.broadcasted_iota(jnp.int32, (G, 2), 1)
    p = jnp.left_shift(jnp.int32(1), lax.rem(d, 16)).astype(jnp.float32)
    return jnp.where(c == d // 16, p, 0.0)


def _body(qf_ref, kf_ref, p0_ref, p1_ref, cand_ref, vals_ref):
    qf = qf_ref[0]
    kf = kf_ref[0]
    projs = (p0_ref[...], p1_ref[...])

    W = _pack_weights()

    qcodes, kcodes, qhash, khash = [], [], [], []
    for g in range(2):
        qg = qf[:, g * G:(g + 1) * G]
        kg = kf[:, g * G:(g + 1) * G]
        qb = (qg > 0).astype(jnp.float32)
        kb = (kg > 0).astype(jnp.float32)
        qcodes.append(lax.dot_general(qb, W, (((1,), (0,)), ((), ())),
                                      preferred_element_type=jnp.float32))
        kcodes.append(lax.dot_general(W, kb, (((0,), (1,)), ((), ())),
                                      preferred_element_type=jnp.float32))
        qy = lax.dot_general(qg, projs[g], (((1,), (0,)), ((), ())),
                             preferred_element_type=jnp.float32)
        ky = lax.dot_general(projs[g], kg, (((0,), (1,)), ((), ())),
                             preferred_element_type=jnp.float32)
        qh = jnp.floor(qy / BW)
        kh = jnp.floor(ky / BW)
        qhash.append(qh - jnp.floor(qh / NB) * NB)
        khash.append(kh - jnp.floor(kh / NB) * NB)

    code_eq = []
    for g in range(2):
        eq = ((qcodes[g][:, 0:1] == kcodes[g][0:1, :]) &
              (qcodes[g][:, 1:2] == kcodes[g][1:2, :]))
        code_eq.append(eq)
    screen = code_eq[0] | code_eq[1]
    any_match = jnp.sum(screen.astype(jnp.int32)) > 0

    cand_ref[...] = jnp.full((1, BQ, K), -1, dtype=jnp.int32)
    vals_ref[...] = jnp.full((1, BQ, K), NEG, dtype=jnp.float32)

    @pl.when(any_match)
    def _heavy():
        full_mask = jnp.zeros((BQ, L), dtype=jnp.bool_)
        for g in range(2):
            lsh = jnp.zeros((BQ, L), dtype=jnp.bool_)
            for h in range(NH):
                lsh = lsh | (qhash[g][:, h:h + 1] == khash[g][h:h + 1, :])
            full_mask = full_mask | (code_eq[g] & lsh)
        scores = lax.dot_general(qf, kf, (((1,), (1,)), ((), ())),
                                 preferred_element_type=jnp.float32)
        masked = jnp.where(full_mask, scores, NEG)
        iota_k = lax.broadcasted_iota(jnp.int32, (BQ, L), 1)
        iota_c = lax.broadcasted_iota(jnp.int32, (BQ, K), 1)

        def step(j, carry):
            m_vals, out_v, out_i = carry
            mx = jnp.max(m_vals, axis=1, keepdims=True)
            idx = jnp.min(jnp.where(m_vals == mx, iota_k, L),
                          axis=1, keepdims=True)
            col = iota_c == j
            out_v = jnp.where(col, mx, out_v)
            out_i = jnp.where(col, jnp.where(mx > NEG, idx, -1), out_i)
            return jnp.where(iota_k == idx, NEG, m_vals), out_v, out_i

        _, out_v, out_i = lax.fori_loop(
            0, K, step,
            (masked,
             jnp.full((BQ, K), NEG, dtype=jnp.float32),
             jnp.full((BQ, K), -1, dtype=jnp.int32)))
        vals_ref[0] = out_v
        cand_ref[0] = out_i


def _run(qf, kf, p0, p1):
    qt = L // BQ
    grid = (B, qt)
    return pl.pallas_call(
        _body,
        grid=grid,
        in_specs=[
            pl.BlockSpec((1, BQ, D), lambda b, t: (b, t, 0)),
            pl.BlockSpec((1, L, D), lambda b, t: (b, 0, 0)),
            pl.BlockSpec((G, NH), lambda b, t: (0, 0)),
            pl.BlockSpec((G, NH), lambda b, t: (0, 0)),
        ],
        out_specs=[
            pl.BlockSpec((1, BQ, K), lambda b, t: (b, t, 0)),
            pl.BlockSpec((1, BQ, K), lambda b, t: (b, t, 0)),
        ],
        out_shape=[
            jax.ShapeDtypeStruct((B, L, K), jnp.int32),
            jax.ShapeDtypeStruct((B, L, K), jnp.float32),
        ],
    )(qf, kf, p0, p1)


def _codes_body(qf_ref, kf_ref, qc0_ref, qc1_ref, kc0_ref, kc1_ref):
    W = _pack_weights()
    for feat_ref, (c0_ref, c1_ref) in ((qf_ref, (qc0_ref, qc1_ref)),
                                       (kf_ref, (kc0_ref, kc1_ref))):
        x = feat_ref[0]
        for g, c_ref in enumerate((c0_ref, c1_ref)):
            xb = (x[:, g * G:(g + 1) * G] > 0).astype(jnp.float32)
            halves = lax.dot_general(W, xb, (((0,), (1,)), ((), ())),
                                     preferred_element_type=jnp.float32)
            code = (halves[0:1, :].astype(jnp.int32) +
                    (halves[1:2, :].astype(jnp.int32) << 16))
            c_ref[0] = code


def _codes(qf, kf):
    return pl.pallas_call(
        _codes_body,
        grid=(B,),
        in_specs=[
            pl.BlockSpec((1, L, D), lambda b: (b, 0, 0)),
            pl.BlockSpec((1, L, D), lambda b: (b, 0, 0)),
        ],
        out_specs=[pl.BlockSpec((1, 1, L), lambda b: (b, 0, 0))] * 4,
        out_shape=[jax.ShapeDtypeStruct((B, 1, L), jnp.int32)] * 4,
    )(qf, kf)


@jax.jit
def _dispatch(qf, kf, p0, p1):
    cand0 = jnp.full((B, L, K), -1, dtype=jnp.int32)
    vals0 = jnp.full((B, L, K), NEG, dtype=jnp.float32)
    qc0, qc1, kc0, kc1 = _codes(qf, kf)
    flags = _make_sc_screen()(qc0, qc1, kc0, kc1)
    any_match = jnp.sum(flags) > 0
    return lax.cond(
        any_match,
        lambda: _run(qf, kf, p0, p1),
        lambda: (cand0, vals0))


def kernel(query_features, key_features, head_idx, lsh_proj_g0, lsh_proj_g1):
    cand, vals = _dispatch(query_features, key_features,
                           lsh_proj_g0, lsh_proj_g1)
    return cand, vals
